# baseline (speedup 1.0000x reference)
import functools

import jax
import jax.numpy as jnp
from jax import lax
from jax.experimental import pallas as pl
from jax.experimental.pallas import tpu as pltpu
from jax.experimental.pallas import tpu_sc as plsc

ROWS = 128
COLS = 100000
LANES = 16
CHUNK = 50000
CHUNKS_PER_ROW = COLS // CHUNK
ROWS_PER_TILE = 4
VECS_PER_CHUNK = CHUNK // LANES
NACC = 5
UNROLL = 5

_BIG_I32 = 2**31 - 1


def _scan_chunk(buf, colbase, accs):
    def body(v, accs):
        accs = list(accs)
        for u in range(NACC):
            x = buf[pl.ds((v + u) * LANES, LANES)]
            bc = jnp.broadcast_to(colbase + (v + u) * LANES, (LANES,))
            bv, bs = accs[u]
            m = x > bv
            accs[u] = (jnp.where(m, x, bv), jnp.where(m, bc, bs))
        return tuple(accs)

    return plsc.parallel_loop(
        0, VECS_PER_CHUNK, step=NACC, unroll=UNROLL, carry=tuple(accs)
    )(body)


@functools.partial(
    pl.kernel,
    out_type=jax.ShapeDtypeStruct((512,), jnp.float32),
    mesh=plsc.VectorSubcoreMesh(core_axis_name="c", subcore_axis_name="s"),
    scratch_types=[
        pltpu.VMEM((CHUNK,), jnp.float32),
        pltpu.VMEM((CHUNK,), jnp.float32),
        pltpu.VMEM((LANES,), jnp.float32),
        pltpu.SemaphoreType.DMA,
        pltpu.SemaphoreType.DMA,
    ],
    compiler_params=pltpu.CompilerParams(
        needs_layout_passes=False,
        use_tc_tiling_on_sc=False,
    ),
)
def _argmax_sc(logits_hbm, out_hbm, buf0, buf1, res_v, sem0, sem1):
    c = lax.axis_index("c")
    s = lax.axis_index("s")
    wid = c * 16 + s
    row0 = wid * ROWS_PER_TILE
    bufs = (buf0, buf1)
    sems = (sem0, sem1)
    n_chunks = ROWS_PER_TILE * CHUNKS_PER_ROW

    def start(t):
        r = t // CHUNKS_PER_ROW
        cc = t % CHUNKS_PER_ROW
        return pltpu.async_copy(
            logits_hbm.at[row0 + r, pl.ds(cc * CHUNK, CHUNK)],
            bufs[t % 2],
            sems[t % 2],
        )

    def fresh():
        return tuple(
            (
                jnp.full((LANES,), -jnp.inf, jnp.float32),
                jnp.zeros((LANES,), jnp.int32),
            )
            for _ in range(NACC)
        )

    copies = [None] * n_chunks
    copies[0] = start(0)

    lane = lax.iota(jnp.int32, LANES)
    res = jnp.zeros((LANES,), jnp.float32)
    accs = fresh()
    for t in range(n_chunks):
        if t + 1 < n_chunks:
            copies[t + 1] = start(t + 1)
        copies[t].wait()
        cc = t % CHUNKS_PER_ROW
        accs = _scan_chunk(bufs[t % 2], jnp.int32(cc * CHUNK), accs)
        if cc == CHUNKS_PER_ROW - 1:
            bv, bi = accs[0][0], accs[0][1] + lane
            for u in range(1, NACC):
                v2, i2 = accs[u][0], accs[u][1] + lane
                better = (v2 > bv) | ((v2 == bv) & (i2 < bi))
                bv = jnp.where(better, v2, bv)
                bi = jnp.where(better, i2, bi)
            m = jnp.max(bv)
            cand = jnp.where(bv == m, bi, jnp.int32(_BIG_I32))
            win = jnp.min(cand).astype(jnp.float32)
            r = t // CHUNKS_PER_ROW
            res = jnp.where(lane == r, win, res)
            accs = fresh()

    res_v[...] = res
    pltpu.sync_copy(res_v, out_hbm.at[pl.ds(wid * LANES, LANES)])


def kernel(logits):
    out = _argmax_sc(logits)
    return out.reshape(32, LANES)[:, :ROWS_PER_TILE].reshape(ROWS)


# speedup vs baseline: 1.7416x; 1.7416x over previous
import functools

import jax
import jax.numpy as jnp
from jax import lax
from jax.experimental import pallas as pl
from jax.experimental.pallas import tpu as pltpu
from jax.experimental.pallas import tpu_sc as plsc

ROWS = 128
COLS = 100000
LANES = 16
TILE_COLS = 128

CHUNK_TILES = 15
CHUNK_COLS = CHUNK_TILES * TILE_COLS
N_CHUNKS = 26
HALF_TILES = CHUNK_TILES * N_CHUNKS
EPI_COL = 2 * HALF_TILES * TILE_COLS
EPI_COLS = COLS - EPI_COL
NBUF = 4

_BIG_I32 = 2**31 - 1


def _scan_chunk(buf, ncols, colbase, accs):
    def body(v, accs):
        accs = list(accs)
        s = jnp.broadcast_to(colbase + v * LANES, (LANES,))
        for r in range(8):
            x = buf[r, pl.ds(v * LANES, LANES)]
            bv, bs = accs[r]
            m = x > bv
            accs[r] = (jnp.where(m, x, bv), jnp.where(m, s, bs))
        return tuple(accs)

    return plsc.parallel_loop(
        0, ncols // LANES, step=1, unroll=2, carry=tuple(accs)
    )(body)


@functools.partial(
    pl.kernel,
    out_type=(
        jax.ShapeDtypeStruct((512,), jnp.float32),
        jax.ShapeDtypeStruct((512,), jnp.int32),
    ),
    mesh=plsc.VectorSubcoreMesh(core_axis_name="c", subcore_axis_name="s"),
    scratch_types=[
        pltpu.VMEM((NBUF, 8, CHUNK_COLS), jnp.float32),
        pltpu.VMEM((8, EPI_COLS), jnp.float32),
        pltpu.VMEM((LANES,), jnp.float32),
        pltpu.VMEM((LANES,), jnp.int32),
        pltpu.SemaphoreType.DMA,
        pltpu.SemaphoreType.DMA,
        pltpu.SemaphoreType.DMA,
        pltpu.SemaphoreType.DMA,
        pltpu.SemaphoreType.DMA,
    ],
    compiler_params=pltpu.CompilerParams(needs_layout_passes=False),
)
def _argmax_sc(
    logits_hbm,
    val_hbm, idx_hbm,
    buf, ebuf,
    stage_v, stage_i,
    sem0, sem1, sem2, sem3, seme,
):
    c = lax.axis_index("c")
    s = lax.axis_index("s")
    rg = c * 8 + lax.rem(s, 8)
    h = s // 8
    row0 = pl.multiple_of(rg * 8, 8)
    sems = (sem0, sem1, sem2, sem3)

    def start(k):
        cb = pl.multiple_of((h * HALF_TILES + k * CHUNK_TILES) * TILE_COLS,
                            TILE_COLS)
        copy = pltpu.async_copy(
            logits_hbm.at[pl.ds(row0, 8), pl.ds(cb, CHUNK_COLS)],
            buf.at[k % NBUF],
            sems[k % NBUF],
        )
        return copy, cb

    epi_copy = pltpu.async_copy(
        logits_hbm.at[pl.ds(row0, 8), pl.ds(EPI_COL, EPI_COLS)], ebuf, seme
    )

    copies = [None] * N_CHUNKS
    cbs = [None] * N_CHUNKS
    for k in range(NBUF - 1):
        copies[k], cbs[k] = start(k)

    accs = tuple(
        (
            jnp.full((LANES,), -jnp.inf, jnp.float32),
            jnp.zeros((LANES,), jnp.int32),
        )
        for _ in range(8)
    )
    for k in range(N_CHUNKS):
        if k + NBUF - 1 < N_CHUNKS:
            copies[k + NBUF - 1], cbs[k + NBUF - 1] = start(k + NBUF - 1)
        copies[k].wait()
        accs = _scan_chunk(buf.at[k % NBUF], CHUNK_COLS, cbs[k], accs)

    epi_copy.wait()
    accs = _scan_chunk(ebuf, EPI_COLS, jnp.int32(EPI_COL), accs)

    lane = lax.iota(jnp.int32, LANES)
    valp = jnp.full((LANES,), -jnp.inf, jnp.float32)
    idxp = jnp.zeros((LANES,), jnp.int32)
    for r in range(8):
        bv, bs = accs[r]
        idx = bs + lane
        m = jnp.max(bv)
        cand = jnp.where(bv == m, idx, jnp.int32(_BIG_I32))
        win = jnp.min(cand)
        valp = jnp.where(lane == r, m, valp)
        idxp = jnp.where(lane == r, win, idxp)

    stage_v[...] = valp
    stage_i[...] = idxp
    wid = c * 16 + s
    pltpu.sync_copy(stage_v, val_hbm.at[pl.ds(wid * LANES, LANES)])
    pltpu.sync_copy(stage_i, idx_hbm.at[pl.ds(wid * LANES, LANES)])


def _merge_tc_body(v_ref, i_ref, o_ref):
    v1, v2 = v_ref[:, 0], v_ref[:, 1]
    i1, i2 = i_ref[:, 0], i_ref[:, 1]
    better = (v2 > v1) | ((v2 == v1) & (i2 < i1))
    o_ref[...] = jnp.where(better, i2, i1).astype(jnp.float32)


_merge_tc = pl.pallas_call(
    _merge_tc_body,
    out_shape=jax.ShapeDtypeStruct((2, 8, 16), jnp.float32),
)


def kernel(logits):
    vals, idxs = _argmax_sc(logits)
    fin = _merge_tc(vals.reshape(2, 2, 8, 16), idxs.reshape(2, 2, 8, 16))
    return fin[:, :, :8].reshape(ROWS)


# speedup vs baseline: 1.7515x; 1.0057x over previous
import functools

import jax
import jax.numpy as jnp
from jax import lax
from jax.experimental import pallas as pl
from jax.experimental.pallas import tpu as pltpu
from jax.experimental.pallas import tpu_sc as plsc

ROWS = 128
COLS = 100000
LANES = 16
TILE_COLS = 128

CHUNK_TILES = 10
CHUNK_COLS = CHUNK_TILES * TILE_COLS
N_CHUNKS = 39
HALF_TILES = CHUNK_TILES * N_CHUNKS
EPI_COL = 2 * HALF_TILES * TILE_COLS
EPI_COLS = COLS - EPI_COL
NBUF = 6

_BIG_I32 = 2**31 - 1


def _scan_chunk(buf, ncols, colbase, accs):
    def body(v, accs):
        accs = list(accs)
        s = jnp.broadcast_to(colbase + v * LANES, (LANES,))
        for r in range(8):
            x = buf[r, pl.ds(v * LANES, LANES)]
            bv, bs = accs[r]
            m = x > bv
            accs[r] = (jnp.where(m, x, bv), jnp.where(m, s, bs))
        return tuple(accs)

    return plsc.parallel_loop(
        0, ncols // LANES, step=1, unroll=2, carry=tuple(accs)
    )(body)


@functools.partial(
    pl.kernel,
    out_type=(
        jax.ShapeDtypeStruct((512,), jnp.float32),
        jax.ShapeDtypeStruct((512,), jnp.int32),
    ),
    mesh=plsc.VectorSubcoreMesh(core_axis_name="c", subcore_axis_name="s"),
    scratch_types=[
        pltpu.VMEM((NBUF, 8, CHUNK_COLS), jnp.float32),
        pltpu.VMEM((8, EPI_COLS), jnp.float32),
        pltpu.VMEM((LANES,), jnp.float32),
        pltpu.VMEM((LANES,), jnp.int32),
        pltpu.SemaphoreType.DMA,
        pltpu.SemaphoreType.DMA,
        pltpu.SemaphoreType.DMA,
        pltpu.SemaphoreType.DMA,
        pltpu.SemaphoreType.DMA,
        pltpu.SemaphoreType.DMA,
        pltpu.SemaphoreType.DMA,
    ],
    compiler_params=pltpu.CompilerParams(needs_layout_passes=False),
)
def _argmax_sc(
    logits_hbm,
    val_hbm, idx_hbm,
    buf, ebuf,
    stage_v, stage_i,
    sem0, sem1, sem2, sem3, sem4, sem5, seme,
):
    c = lax.axis_index("c")
    s = lax.axis_index("s")
    rg = c * 8 + lax.rem(s, 8)
    h = s // 8
    row0 = pl.multiple_of(rg * 8, 8)
    sems = (sem0, sem1, sem2, sem3, sem4, sem5)

    def start(k):
        cb = pl.multiple_of((h * HALF_TILES + k * CHUNK_TILES) * TILE_COLS,
                            TILE_COLS)
        copy = pltpu.async_copy(
            logits_hbm.at[pl.ds(row0, 8), pl.ds(cb, CHUNK_COLS)],
            buf.at[k % NBUF],
            sems[k % NBUF],
        )
        return copy, cb

    epi_copy = pltpu.async_copy(
        logits_hbm.at[pl.ds(row0, 8), pl.ds(EPI_COL, EPI_COLS)], ebuf, seme
    )

    copies = [None] * N_CHUNKS
    cbs = [None] * N_CHUNKS
    for k in range(NBUF - 1):
        copies[k], cbs[k] = start(k)

    accs = tuple(
        (
            jnp.full((LANES,), -jnp.inf, jnp.float32),
            jnp.zeros((LANES,), jnp.int32),
        )
        for _ in range(8)
    )
    for k in range(N_CHUNKS):
        if k + NBUF - 1 < N_CHUNKS:
            copies[k + NBUF - 1], cbs[k + NBUF - 1] = start(k + NBUF - 1)
        copies[k].wait()
        accs = _scan_chunk(buf.at[k % NBUF], CHUNK_COLS, cbs[k], accs)

    epi_copy.wait()
    accs = _scan_chunk(ebuf, EPI_COLS, jnp.int32(EPI_COL), accs)

    lane = lax.iota(jnp.int32, LANES)
    valp = jnp.full((LANES,), -jnp.inf, jnp.float32)
    idxp = jnp.zeros((LANES,), jnp.int32)
    for r in range(8):
        bv, bs = accs[r]
        idx = bs + lane
        m = jnp.max(bv)
        cand = jnp.where(bv == m, idx, jnp.int32(_BIG_I32))
        win = jnp.min(cand)
        valp = jnp.where(lane == r, m, valp)
        idxp = jnp.where(lane == r, win, idxp)

    stage_v[...] = valp
    stage_i[...] = idxp
    wid = c * 16 + s
    pltpu.sync_copy(stage_v, val_hbm.at[pl.ds(wid * LANES, LANES)])
    pltpu.sync_copy(stage_i, idx_hbm.at[pl.ds(wid * LANES, LANES)])


def _merge_tc_body(v_ref, i_ref, o_ref):
    v1, v2 = v_ref[:, 0], v_ref[:, 1]
    i1, i2 = i_ref[:, 0], i_ref[:, 1]
    better = (v2 > v1) | ((v2 == v1) & (i2 < i1))
    o_ref[...] = jnp.where(better, i2, i1).astype(jnp.float32)


_merge_tc = pl.pallas_call(
    _merge_tc_body,
    out_shape=jax.ShapeDtypeStruct((2, 8, 16), jnp.float32),
)


def kernel(logits):
    vals, idxs = _argmax_sc(logits)
    fin = _merge_tc(vals.reshape(2, 2, 8, 16), idxs.reshape(2, 2, 8, 16))
    return fin[:, :, :8].reshape(ROWS)
